# baseline (speedup 1.0000x reference)
import jax
import jax.numpy as jnp
from jax import lax
from jax.experimental import pallas as pl
from jax.experimental.pallas import tpu as pltpu
from jax.experimental.pallas import tpu_sc as plsc

_B = 1024
_L = 16
_CHUNK = 128
_NCHUNKS = _B // _CHUNK
_JPC = _CHUNK // _L


def _sc_body(logits_hbm, target_hbm, out_hbm, idx_v, vals_v, sem):
    wid = lax.axis_index("s")

    @pl.when(wid == 0)
    def _():
        pltpu.sync_copy(target_hbm, idx_v)

        lane = lax.iota(jnp.int32, _L)
        copies = []
        for k in range(_NCHUNKS):
            for jj in range(_JPC):
                j = k * _JPC + jj
                rc = ((j * _L) >> 7) * 1024 + ((j * _L) & 127)
                t16 = idx_v[pl.ds(j * _L, _L)]
                addr = ((t16 >> 3) << 13) + ((t16 & 7) << 7) + (lane + rc)
                idx_v[pl.ds(j * _L, _L)] = addr
            copies.append(pltpu.async_copy(
                logits_hbm.at[idx_v.at[pl.ds(k * _CHUNK, _CHUNK)]],
                vals_v.at[pl.ds(k * _CHUNK, _CHUNK)],
                sem,
            ))

        p = None
        for k in range(_NCHUNKS):
            copies[k].wait()
            for jj in range(_JPC):
                j = k * _JPC + jj
                v = vals_v[pl.ds(j * _L, _L)]
                p = v if p is None else p + v

        s = p[0]
        for j in range(1, _L):
            s = s + p[j]
        vals_v[pl.ds(0, _L)] = jnp.broadcast_to(-s, (_L,))
        pltpu.sync_copy(vals_v.at[pl.ds(0, _L)], out_hbm)


def kernel(logits, target):
    flat = (
        logits.reshape(8, 128, 12500, 8).transpose(2, 0, 3, 1).reshape(-1)
    )
    tgt = target.astype(jnp.int32)

    fn = pl.kernel(
        _sc_body,
        mesh=plsc.VectorSubcoreMesh(
            core_axis_name="c", subcore_axis_name="s", num_cores=1),
        out_type=jax.ShapeDtypeStruct((_L,), jnp.float32),
        scratch_types=[
            pltpu.VMEM((_B,), jnp.int32),
            pltpu.VMEM((_B,), jnp.float32),
            pltpu.SemaphoreType.DMA,
        ],
    )
    out = fn(flat, tgt)
    return out[0]


# speedup vs baseline: 1.0119x; 1.0119x over previous
import jax
import jax.numpy as jnp
from jax import lax
from jax.experimental import pallas as pl
from jax.experimental.pallas import tpu as pltpu
from jax.experimental.pallas import tpu_sc as plsc

_B = 1024
_L = 16
_CHUNK = 128
_NCHUNKS = _B // _CHUNK


def _sc_body(logits_hbm, target_hbm, out_hbm, idx_v, vals_v, sem):
    wid = lax.axis_index("s")

    @pl.when(wid == 0)
    def _():
        pltpu.sync_copy(target_hbm, idx_v)

        lane = lax.iota(jnp.int32, _L)
        for j in range(_B // _L):
            t16 = idx_v[pl.ds(j * _L, _L)]
            rows = (j * _L) + lane
            addr = (
                ((t16 >> 3) << 13)
                + ((rows >> 7) << 10)
                + ((t16 & 7) << 7)
                + (rows & 127)
            )
            idx_v[pl.ds(j * _L, _L)] = addr

        copies = []
        for k in range(_NCHUNKS):
            copies.append(pltpu.async_copy(
                logits_hbm.at[idx_v.at[pl.ds(k * _CHUNK, _CHUNK)]],
                vals_v.at[pl.ds(k * _CHUNK, _CHUNK)],
                sem,
            ))
        for c in copies:
            c.wait()

        p0 = vals_v[pl.ds(0, _L)]
        p1 = vals_v[pl.ds(_L, _L)]
        for j in range(2, _B // _L, 2):
            p0 = p0 + vals_v[pl.ds(j * _L, _L)]
            p1 = p1 + vals_v[pl.ds((j + 1) * _L, _L)]
        p = p0 + p1
        lane_u = lax.iota(jnp.int32, _L)
        for sh in (8, 4, 2, 1):
            p = p + p.at[lane_u ^ sh].get(mode="promise_in_bounds")
        vals_v[pl.ds(0, _L)] = -p
        pltpu.sync_copy(vals_v.at[pl.ds(0, _L)], out_hbm)


def kernel(logits, target):
    flat = (
        logits.reshape(8, 128, 12500, 8).transpose(2, 0, 3, 1).reshape(-1)
    )
    tgt = target.astype(jnp.int32)

    fn = pl.kernel(
        _sc_body,
        mesh=plsc.VectorSubcoreMesh(
            core_axis_name="c", subcore_axis_name="s", num_cores=1),
        out_type=jax.ShapeDtypeStruct((_L,), jnp.float32),
        scratch_types=[
            pltpu.VMEM((_B,), jnp.int32),
            pltpu.VMEM((_B,), jnp.float32),
            pltpu.SemaphoreType.DMA,
        ],
    )
    out = fn(flat, tgt)
    return out[0]
